# baseline (speedup 1.0000x reference)
import jax
import jax.numpy as jnp
from jax import lax
from jax.experimental import pallas as pl
from jax.experimental.pallas import tpu as pltpu
from jax.experimental.pallas import tpu_sc as plsc

NUM_SEG = 512
SEG_PAD = 513
NE = 3_200_000
NN = 100_000

NC, NS, L = 2, 16, 16
NW = NC * NS

E_PER_W = NE // NW
CE = 4_000
N_CHUNKS = E_PER_W // CE
EB = 800

NPW = 3_136
NPW_LAST = NN - (NW - 1) * NPW


def _sc_body(pq_hbm, tq_hbm, eg_hbm, ex_hbm, t2_hbm, p2_hbm, ng_hbm, out_hbm,
             bpq, btq, bsg, nex, nt2, np2, nng, tab, res, sem_n, sem_e0, sem_e1):
    wid = lax.axis_index("s") * NC + lax.axis_index("c")
    lane = lax.iota(jnp.int32, L)
    zero = jnp.zeros((L,), jnp.float32)
    ones = jnp.ones((L,), jnp.float32)

    def q_idx(q):
        return jnp.full((L,), q, jnp.int32)

    ebase = wid * E_PER_W
    esems = (sem_e0, sem_e1)

    def e_start(k, b):
        off = ebase + k * CE
        pltpu.async_copy(pq_hbm.at[pl.ds(off, CE)], bpq.at[b], esems[b])
        pltpu.async_copy(tq_hbm.at[pl.ds(off, CE)], btq.at[b], esems[b])
        pltpu.async_copy(eg_hbm.at[pl.ds(off, CE)], bsg.at[b], esems[b])

    def e_wait(b):
        pltpu.make_async_copy(pq_hbm.at[pl.ds(0, CE)], bpq.at[b], esems[b]).wait()
        pltpu.make_async_copy(tq_hbm.at[pl.ds(0, CE)], btq.at[b], esems[b]).wait()
        pltpu.make_async_copy(eg_hbm.at[pl.ds(0, CE)], bsg.at[b], esems[b]).wait()

    def e_compute(b):
        def eblock(i, _):
            bo = i * EB
            sg_first = bsg[b, pl.ds(bo, L)]
            sg_last = bsg[b, pl.ds(bo + EB - L, L)]
            uniform = jnp.all(sg_first == sg_last)

            def fast():
                a1 = a2 = a3 = zero
                for t in range(EB // L):
                    o = bo + t * L
                    pq = bpq[b, pl.ds(o, L)]
                    tq = btq[b, pl.ds(o, L)]
                    d = pq - tq
                    a1 = a1 + d * d
                    a2 = a2 + tq * tq
                    a3 = a3 + pq * pq
                plsc.addupdate_scatter(tab, [q_idx(0), lane, sg_first], a1)
                plsc.addupdate_scatter(tab, [q_idx(1), lane, sg_first], a2)
                plsc.addupdate_scatter(tab, [q_idx(2), lane, sg_first], a3)

            def slow():
                def ebody(t, _):
                    o = bo + t * L
                    pq = bpq[b, pl.ds(o, L)]
                    tq = btq[b, pl.ds(o, L)]
                    sg = bsg[b, pl.ds(o, L)]
                    d = pq - tq
                    plsc.addupdate_scatter(tab, [q_idx(0), lane, sg], d * d)
                    plsc.addupdate_scatter(tab, [q_idx(1), lane, sg], tq * tq)
                    plsc.addupdate_scatter(tab, [q_idx(2), lane, sg], pq * pq)
                    return 0

                lax.fori_loop(0, EB // L, ebody, 0)

            lax.cond(uniform, fast, slow)
            return 0

        lax.fori_loop(0, CE // EB, eblock, 0)

    e_start(0, 0)
    e_start(1, 1)

    nbase = jnp.minimum(wid * NPW, NN - NPW)
    h_ex = pltpu.async_copy(ex_hbm.at[pl.ds(nbase, NPW)], nex, sem_n)
    h_t2 = pltpu.async_copy(t2_hbm.at[pl.ds(nbase, NPW)], nt2, sem_n)
    h_p2 = pltpu.async_copy(p2_hbm.at[pl.ds(nbase, NPW)], np2, sem_n)
    h_ng = pltpu.async_copy(ng_hbm.at[pl.ds(nbase, NPW)], nng, sem_n)

    def zbody(r, _):
        for q in range(7):
            for l in range(L):
                tab[q, l, pl.ds(r * L, L)] = zero
        return 0

    lax.fori_loop(0, SEG_PAD // L, zbody, 0)
    for q in range(7):
        for l in range(L):
            tab[q, l, pl.ds(SEG_PAD - L, L)] = zero

    def pair(j, _):
        k0 = 2 * j

        e_wait(0)
        e_compute(0)

        @pl.when(k0 + 2 < N_CHUNKS)
        def _s0():
            e_start(k0 + 2, 0)

        e_wait(1)
        e_compute(1)

        @pl.when(k0 + 3 < N_CHUNKS)
        def _s1():
            e_start(k0 + 3, 1)

        return 0

    lax.fori_loop(0, N_CHUNKS // 2, pair, 0)
    if N_CHUNKS % 2:
        e_wait(0)
        e_compute(0)

    h_ex.wait()
    h_t2.wait()
    h_p2.wait()
    h_ng.wait()

    is_last = wid == NW - 1
    so = jnp.where(is_last, NPW - NPW_LAST, 0)
    n_it = jnp.where(is_last, NPW_LAST // L, NPW // L)

    def nbody(i, _):
        o = so + i * L
        ex = nex[pl.ds(o, L)]
        t2 = nt2[pl.ds(o, L)]
        p2 = np2[pl.ds(o, L)]
        sg = nng[pl.ds(o, L)]
        plsc.addupdate_scatter(tab, [q_idx(3), lane, sg], ex)
        plsc.addupdate_scatter(tab, [q_idx(4), lane, sg], t2)
        plsc.addupdate_scatter(tab, [q_idx(5), lane, sg], p2)
        plsc.addupdate_scatter(tab, [q_idx(6), lane, sg], ones)
        return 0

    lax.fori_loop(0, n_it, nbody, 0)

    for q in range(7):
        def lred(j, _):
            o = j * L
            acc = tab[q, 0, pl.ds(o, L)]
            for l in range(1, L):
                acc = acc + tab[q, l, pl.ds(o, L)]
            res[pl.ds(q * NUM_SEG + o, L)] = acc
            return 0

        lax.fori_loop(0, NUM_SEG // L, lred, 0)

    pltpu.sync_copy(res, out_hbm.at[pl.ds(wid * 7 * NUM_SEG, 7 * NUM_SEG)])


def _sc_accumulate(pq, tq, eg, ex, t2, p2, ng):
    mesh = plsc.VectorSubcoreMesh(
        core_axis_name="c", subcore_axis_name="s", num_cores=NC, num_subcores=NS
    )
    f = pl.kernel(
        _sc_body,
        out_type=jax.ShapeDtypeStruct((NW * 7 * NUM_SEG,), jnp.float32),
        mesh=mesh,
        scratch_types=[
            pltpu.VMEM((2, CE), jnp.float32),
            pltpu.VMEM((2, CE), jnp.float32),
            pltpu.VMEM((2, CE), jnp.int32),
            pltpu.VMEM((NPW,), jnp.float32),
            pltpu.VMEM((NPW,), jnp.float32),
            pltpu.VMEM((NPW,), jnp.float32),
            pltpu.VMEM((NPW,), jnp.int32),
            pltpu.VMEM((7, L, SEG_PAD), jnp.float32),
            pltpu.VMEM((7 * NUM_SEG,), jnp.float32),
            pltpu.SemaphoreType.DMA,
            pltpu.SemaphoreType.DMA,
            pltpu.SemaphoreType.DMA,
        ],
        compiler_params=pltpu.CompilerParams(
            use_tc_tiling_on_sc=False, needs_layout_passes=False
        ),
    )
    return f(pq, tq, eg, ex, t2, p2, ng)


def _epi_body(t_ref, o_ref):
    t = t_ref[...]
    s = jnp.sum(t.reshape(NW, 28, 128), axis=0)
    cnt = s[24:28]
    nerr = jnp.sqrt(s[0:4])
    denq = jnp.sqrt(s[4:8])
    psq = jnp.sqrt(s[8:12])
    perrq = nerr / denq
    rmsd = jnp.sqrt(s[12:16] / cnt)
    denx = jnp.sqrt(s[16:20] / cnt)
    psx = jnp.sqrt(s[20:24] / cnt)
    perrx = rmsd / denx
    out8 = jnp.stack(
        [rmsd, perrx, psx, denx, nerr, perrq, psq, denq]
    )
    o_ref[...] = jnp.sum(out8, axis=(1, 2))


def _tc_epilogue(part):
    return pl.pallas_call(
        _epi_body,
        out_shape=jax.ShapeDtypeStruct((8,), jnp.float32),
    )(part)


def kernel(pred_x, pred_q, target_x, target_q, edge2graph, node2graph,
           atom_type, edge_r, edge_p):
    del atom_type, edge_r, edge_p
    d = pred_x - target_x
    ex = jnp.sum(d * d, axis=-1)
    t2 = jnp.sum(target_x * target_x, axis=-1)
    p2 = jnp.sum(pred_x * pred_x, axis=-1)
    part = _sc_accumulate(
        pred_q, target_q, edge2graph.astype(jnp.int32),
        ex, t2, p2, node2graph.astype(jnp.int32))
    return _tc_epilogue(part.reshape(NW * 28, 128))


# speedup vs baseline: 1.0428x; 1.0428x over previous
import jax
import jax.numpy as jnp
from jax import lax
from jax.experimental import pallas as pl
from jax.experimental.pallas import tpu as pltpu
from jax.experimental.pallas import tpu_sc as plsc

NUM_SEG = 512
SEG_PAD = 513
NE = 3_200_000
NN = 100_000

NC, NS, L = 2, 16, 16
NW = NC * NS

E_PER_W = NE // NW
CE = 4_000
N_CHUNKS = E_PER_W // CE
EB = 400

NPW = 3_136
NPW_LAST = NN - (NW - 1) * NPW


def _sc_body(pq_hbm, tq_hbm, eg_hbm, ex_hbm, t2_hbm, p2_hbm, ng_hbm, out_hbm,
             bpq, btq, bsg, nex, nt2, np2, nng, tab, res, sem_n, sem_e0, sem_e1):
    wid = lax.axis_index("s") * NC + lax.axis_index("c")
    lane = lax.iota(jnp.int32, L)
    zero = jnp.zeros((L,), jnp.float32)
    ones = jnp.ones((L,), jnp.float32)

    def q_idx(q):
        return jnp.full((L,), q, jnp.int32)

    ebase = wid * E_PER_W
    esems = (sem_e0, sem_e1)

    def e_start(k, b):
        off = ebase + k * CE
        pltpu.async_copy(pq_hbm.at[pl.ds(off, CE)], bpq.at[b], esems[b])
        pltpu.async_copy(tq_hbm.at[pl.ds(off, CE)], btq.at[b], esems[b])
        pltpu.async_copy(eg_hbm.at[pl.ds(off, CE)], bsg.at[b], esems[b])

    def e_wait(b):
        pltpu.make_async_copy(pq_hbm.at[pl.ds(0, CE)], bpq.at[b], esems[b]).wait()
        pltpu.make_async_copy(tq_hbm.at[pl.ds(0, CE)], btq.at[b], esems[b]).wait()
        pltpu.make_async_copy(eg_hbm.at[pl.ds(0, CE)], bsg.at[b], esems[b]).wait()

    def e_compute(b):
        def eblock(i, _):
            bo = i * EB
            sg_first = bsg[b, pl.ds(bo, L)]
            sg_last = bsg[b, pl.ds(bo + EB - L, L)]
            uniform = jnp.all(sg_first == sg_last)

            def fast():
                a1 = a2 = a3 = zero
                for t in range(EB // L):
                    o = bo + t * L
                    pq = bpq[b, pl.ds(o, L)]
                    tq = btq[b, pl.ds(o, L)]
                    d = pq - tq
                    a1 = a1 + d * d
                    a2 = a2 + tq * tq
                    a3 = a3 + pq * pq
                plsc.addupdate_scatter(tab, [q_idx(0), lane, sg_first], a1)
                plsc.addupdate_scatter(tab, [q_idx(1), lane, sg_first], a2)
                plsc.addupdate_scatter(tab, [q_idx(2), lane, sg_first], a3)

            def slow():
                def ebody(t, _):
                    o = bo + t * L
                    pq = bpq[b, pl.ds(o, L)]
                    tq = btq[b, pl.ds(o, L)]
                    sg = bsg[b, pl.ds(o, L)]
                    d = pq - tq
                    plsc.addupdate_scatter(tab, [q_idx(0), lane, sg], d * d)
                    plsc.addupdate_scatter(tab, [q_idx(1), lane, sg], tq * tq)
                    plsc.addupdate_scatter(tab, [q_idx(2), lane, sg], pq * pq)
                    return 0

                lax.fori_loop(0, EB // L, ebody, 0)

            lax.cond(uniform, fast, slow)
            return 0

        lax.fori_loop(0, CE // EB, eblock, 0)

    e_start(0, 0)
    e_start(1, 1)

    nbase = jnp.minimum(wid * NPW, NN - NPW)
    h_ex = pltpu.async_copy(ex_hbm.at[pl.ds(nbase, NPW)], nex, sem_n)
    h_t2 = pltpu.async_copy(t2_hbm.at[pl.ds(nbase, NPW)], nt2, sem_n)
    h_p2 = pltpu.async_copy(p2_hbm.at[pl.ds(nbase, NPW)], np2, sem_n)
    h_ng = pltpu.async_copy(ng_hbm.at[pl.ds(nbase, NPW)], nng, sem_n)

    def zbody(r, _):
        for q in range(7):
            for l in range(L):
                tab[q, l, pl.ds(r * L, L)] = zero
        return 0

    lax.fori_loop(0, SEG_PAD // L, zbody, 0)
    for q in range(7):
        for l in range(L):
            tab[q, l, pl.ds(SEG_PAD - L, L)] = zero

    def pair(j, _):
        k0 = 2 * j

        e_wait(0)
        e_compute(0)

        @pl.when(k0 + 2 < N_CHUNKS)
        def _s0():
            e_start(k0 + 2, 0)

        e_wait(1)
        e_compute(1)

        @pl.when(k0 + 3 < N_CHUNKS)
        def _s1():
            e_start(k0 + 3, 1)

        return 0

    lax.fori_loop(0, N_CHUNKS // 2, pair, 0)
    if N_CHUNKS % 2:
        e_wait(0)
        e_compute(0)

    h_ex.wait()
    h_t2.wait()
    h_p2.wait()
    h_ng.wait()

    is_last = wid == NW - 1
    so = jnp.where(is_last, NPW - NPW_LAST, 0)
    n_it = jnp.where(is_last, NPW_LAST // L, NPW // L)

    def nbody(i, _):
        o = so + i * L
        ex = nex[pl.ds(o, L)]
        t2 = nt2[pl.ds(o, L)]
        p2 = np2[pl.ds(o, L)]
        sg = nng[pl.ds(o, L)]
        plsc.addupdate_scatter(tab, [q_idx(3), lane, sg], ex)
        plsc.addupdate_scatter(tab, [q_idx(4), lane, sg], t2)
        plsc.addupdate_scatter(tab, [q_idx(5), lane, sg], p2)
        plsc.addupdate_scatter(tab, [q_idx(6), lane, sg], ones)
        return 0

    lax.fori_loop(0, n_it, nbody, 0)

    for q in range(7):
        def lred(j, _):
            o = j * L
            acc = tab[q, 0, pl.ds(o, L)]
            for l in range(1, L):
                acc = acc + tab[q, l, pl.ds(o, L)]
            res[pl.ds(q * NUM_SEG + o, L)] = acc
            return 0

        lax.fori_loop(0, NUM_SEG // L, lred, 0)

    pltpu.sync_copy(res, out_hbm.at[pl.ds(wid * 7 * NUM_SEG, 7 * NUM_SEG)])


def _sc_accumulate(pq, tq, eg, ex, t2, p2, ng):
    mesh = plsc.VectorSubcoreMesh(
        core_axis_name="c", subcore_axis_name="s", num_cores=NC, num_subcores=NS
    )
    f = pl.kernel(
        _sc_body,
        out_type=jax.ShapeDtypeStruct((NW * 7 * NUM_SEG,), jnp.float32),
        mesh=mesh,
        scratch_types=[
            pltpu.VMEM((2, CE), jnp.float32),
            pltpu.VMEM((2, CE), jnp.float32),
            pltpu.VMEM((2, CE), jnp.int32),
            pltpu.VMEM((NPW,), jnp.float32),
            pltpu.VMEM((NPW,), jnp.float32),
            pltpu.VMEM((NPW,), jnp.float32),
            pltpu.VMEM((NPW,), jnp.int32),
            pltpu.VMEM((7, L, SEG_PAD), jnp.float32),
            pltpu.VMEM((7 * NUM_SEG,), jnp.float32),
            pltpu.SemaphoreType.DMA,
            pltpu.SemaphoreType.DMA,
            pltpu.SemaphoreType.DMA,
        ],
        compiler_params=pltpu.CompilerParams(
            use_tc_tiling_on_sc=False, needs_layout_passes=False
        ),
    )
    return f(pq, tq, eg, ex, t2, p2, ng)


def _epi_body(t_ref, o_ref):
    t = t_ref[...]
    s = jnp.sum(t.reshape(NW, 28, 128), axis=0)
    cnt = s[24:28]
    nerr = jnp.sqrt(s[0:4])
    denq = jnp.sqrt(s[4:8])
    psq = jnp.sqrt(s[8:12])
    perrq = nerr / denq
    rmsd = jnp.sqrt(s[12:16] / cnt)
    denx = jnp.sqrt(s[16:20] / cnt)
    psx = jnp.sqrt(s[20:24] / cnt)
    perrx = rmsd / denx
    out8 = jnp.stack(
        [rmsd, perrx, psx, denx, nerr, perrq, psq, denq]
    )
    o_ref[...] = jnp.sum(out8, axis=(1, 2))


def _tc_epilogue(part):
    return pl.pallas_call(
        _epi_body,
        out_shape=jax.ShapeDtypeStruct((8,), jnp.float32),
    )(part)


def kernel(pred_x, pred_q, target_x, target_q, edge2graph, node2graph,
           atom_type, edge_r, edge_p):
    del atom_type, edge_r, edge_p
    d = pred_x - target_x
    ex = jnp.sum(d * d, axis=-1)
    t2 = jnp.sum(target_x * target_x, axis=-1)
    p2 = jnp.sum(pred_x * pred_x, axis=-1)
    part = _sc_accumulate(
        pred_q, target_q, edge2graph.astype(jnp.int32),
        ex, t2, p2, node2graph.astype(jnp.int32))
    return _tc_epilogue(part.reshape(NW * 28, 128))


# speedup vs baseline: 1.0475x; 1.0045x over previous
import jax
import jax.numpy as jnp
from jax import lax
from jax.experimental import pallas as pl
from jax.experimental.pallas import tpu as pltpu
from jax.experimental.pallas import tpu_sc as plsc

NUM_SEG = 512
SEG_PAD = 513
NE = 3_200_000
NN = 100_000

NC, NS, L = 2, 16, 16
NW = NC * NS

E_PER_W = NE // NW
CE = 2_000
N_CHUNKS = E_PER_W // CE
NB = 3
EB = 400

NPW = 3_136
NPW_LAST = NN - (NW - 1) * NPW


def _sc_body(pq_hbm, tq_hbm, eg_hbm, ex_hbm, t2_hbm, p2_hbm, ng_hbm, out_hbm,
             bpq, btq, bsg, nex, nt2, np2, nng, tab, res, sem_n, sem_e0,
             sem_e1, sem_e2):
    wid = lax.axis_index("s") * NC + lax.axis_index("c")
    lane = lax.iota(jnp.int32, L)
    zero = jnp.zeros((L,), jnp.float32)
    ones = jnp.ones((L,), jnp.float32)

    def q_idx(q):
        return jnp.full((L,), q, jnp.int32)

    ebase = wid * E_PER_W
    esems = (sem_e0, sem_e1, sem_e2)

    def e_start(k, b):
        off = ebase + k * CE
        pltpu.async_copy(pq_hbm.at[pl.ds(off, CE)], bpq.at[b], esems[b])
        pltpu.async_copy(tq_hbm.at[pl.ds(off, CE)], btq.at[b], esems[b])
        pltpu.async_copy(eg_hbm.at[pl.ds(off, CE)], bsg.at[b], esems[b])

    def e_wait(b):
        pltpu.make_async_copy(pq_hbm.at[pl.ds(0, CE)], bpq.at[b], esems[b]).wait()
        pltpu.make_async_copy(tq_hbm.at[pl.ds(0, CE)], btq.at[b], esems[b]).wait()
        pltpu.make_async_copy(eg_hbm.at[pl.ds(0, CE)], bsg.at[b], esems[b]).wait()

    def e_compute(b):
        def eblock(i, _):
            bo = i * EB
            sg_first = bsg[b, pl.ds(bo, L)]
            sg_last = bsg[b, pl.ds(bo + EB - L, L)]
            uniform = jnp.all(sg_first == sg_last)

            def fast():
                a1 = a2 = a3 = zero
                for t in range(EB // L):
                    o = bo + t * L
                    pq = bpq[b, pl.ds(o, L)]
                    tq = btq[b, pl.ds(o, L)]
                    d = pq - tq
                    a1 = a1 + d * d
                    a2 = a2 + tq * tq
                    a3 = a3 + pq * pq
                plsc.addupdate_scatter(tab, [q_idx(0), lane, sg_first], a1)
                plsc.addupdate_scatter(tab, [q_idx(1), lane, sg_first], a2)
                plsc.addupdate_scatter(tab, [q_idx(2), lane, sg_first], a3)

            def slow():
                def ebody(t, _):
                    o = bo + t * L
                    pq = bpq[b, pl.ds(o, L)]
                    tq = btq[b, pl.ds(o, L)]
                    sg = bsg[b, pl.ds(o, L)]
                    d = pq - tq
                    plsc.addupdate_scatter(tab, [q_idx(0), lane, sg], d * d)
                    plsc.addupdate_scatter(tab, [q_idx(1), lane, sg], tq * tq)
                    plsc.addupdate_scatter(tab, [q_idx(2), lane, sg], pq * pq)
                    return 0

                lax.fori_loop(0, EB // L, ebody, 0)

            lax.cond(uniform, fast, slow)
            return 0

        lax.fori_loop(0, CE // EB, eblock, 0)

    for b0 in range(NB):
        e_start(b0, b0)

    nbase = jnp.minimum(wid * NPW, NN - NPW)
    h_ex = pltpu.async_copy(ex_hbm.at[pl.ds(nbase, NPW)], nex, sem_n)
    h_t2 = pltpu.async_copy(t2_hbm.at[pl.ds(nbase, NPW)], nt2, sem_n)
    h_p2 = pltpu.async_copy(p2_hbm.at[pl.ds(nbase, NPW)], np2, sem_n)
    h_ng = pltpu.async_copy(ng_hbm.at[pl.ds(nbase, NPW)], nng, sem_n)

    def zbody(r, _):
        for q in range(7):
            for l in range(L):
                tab[q, l, pl.ds(r * L, L)] = zero
        return 0

    lax.fori_loop(0, SEG_PAD // L, zbody, 0)
    for q in range(7):
        for l in range(L):
            tab[q, l, pl.ds(SEG_PAD - L, L)] = zero

    def ring(j, _):
        k0 = NB * j
        for b in range(NB):

            e_wait(b)
            e_compute(b)

            @pl.when(k0 + b + NB < N_CHUNKS)
            def _s():
                e_start(k0 + b + NB, b)

        return 0

    lax.fori_loop(0, N_CHUNKS // NB, ring, 0)
    for r in range(N_CHUNKS % NB):
        e_wait(r)
        e_compute(r)

    h_ex.wait()
    h_t2.wait()
    h_p2.wait()
    h_ng.wait()

    is_last = wid == NW - 1
    so = jnp.where(is_last, NPW - NPW_LAST, 0)
    n_it = jnp.where(is_last, NPW_LAST // L, NPW // L)

    def nbody(i, _):
        o = so + i * L
        ex = nex[pl.ds(o, L)]
        t2 = nt2[pl.ds(o, L)]
        p2 = np2[pl.ds(o, L)]
        sg = nng[pl.ds(o, L)]
        plsc.addupdate_scatter(tab, [q_idx(3), lane, sg], ex)
        plsc.addupdate_scatter(tab, [q_idx(4), lane, sg], t2)
        plsc.addupdate_scatter(tab, [q_idx(5), lane, sg], p2)
        plsc.addupdate_scatter(tab, [q_idx(6), lane, sg], ones)
        return 0

    lax.fori_loop(0, n_it, nbody, 0)

    for q in range(7):
        def lred(j, _):
            o = j * L
            acc = tab[q, 0, pl.ds(o, L)]
            for l in range(1, L):
                acc = acc + tab[q, l, pl.ds(o, L)]
            res[pl.ds(q * NUM_SEG + o, L)] = acc
            return 0

        lax.fori_loop(0, NUM_SEG // L, lred, 0)

    pltpu.sync_copy(res, out_hbm.at[pl.ds(wid * 7 * NUM_SEG, 7 * NUM_SEG)])


def _sc_accumulate(pq, tq, eg, ex, t2, p2, ng):
    mesh = plsc.VectorSubcoreMesh(
        core_axis_name="c", subcore_axis_name="s", num_cores=NC, num_subcores=NS
    )
    f = pl.kernel(
        _sc_body,
        out_type=jax.ShapeDtypeStruct((NW * 7 * NUM_SEG,), jnp.float32),
        mesh=mesh,
        scratch_types=[
            pltpu.VMEM((NB, CE), jnp.float32),
            pltpu.VMEM((NB, CE), jnp.float32),
            pltpu.VMEM((NB, CE), jnp.int32),
            pltpu.VMEM((NPW,), jnp.float32),
            pltpu.VMEM((NPW,), jnp.float32),
            pltpu.VMEM((NPW,), jnp.float32),
            pltpu.VMEM((NPW,), jnp.int32),
            pltpu.VMEM((7, L, SEG_PAD), jnp.float32),
            pltpu.VMEM((7 * NUM_SEG,), jnp.float32),
            pltpu.SemaphoreType.DMA,
            pltpu.SemaphoreType.DMA,
            pltpu.SemaphoreType.DMA,
            pltpu.SemaphoreType.DMA,
        ],
        compiler_params=pltpu.CompilerParams(
            use_tc_tiling_on_sc=False, needs_layout_passes=False
        ),
    )
    return f(pq, tq, eg, ex, t2, p2, ng)


def _epi_body(t_ref, o_ref):
    t = t_ref[...]
    s = jnp.sum(t.reshape(NW, 28, 128), axis=0)
    cnt = s[24:28]
    nerr = jnp.sqrt(s[0:4])
    denq = jnp.sqrt(s[4:8])
    psq = jnp.sqrt(s[8:12])
    perrq = nerr / denq
    rmsd = jnp.sqrt(s[12:16] / cnt)
    denx = jnp.sqrt(s[16:20] / cnt)
    psx = jnp.sqrt(s[20:24] / cnt)
    perrx = rmsd / denx
    out8 = jnp.stack(
        [rmsd, perrx, psx, denx, nerr, perrq, psq, denq]
    )
    o_ref[...] = jnp.sum(out8, axis=(1, 2))


def _tc_epilogue(part):
    return pl.pallas_call(
        _epi_body,
        out_shape=jax.ShapeDtypeStruct((8,), jnp.float32),
    )(part)


def kernel(pred_x, pred_q, target_x, target_q, edge2graph, node2graph,
           atom_type, edge_r, edge_p):
    del atom_type, edge_r, edge_p
    d = pred_x - target_x
    ex = jnp.sum(d * d, axis=-1)
    t2 = jnp.sum(target_x * target_x, axis=-1)
    p2 = jnp.sum(pred_x * pred_x, axis=-1)
    part = _sc_accumulate(
        pred_q, target_q, edge2graph.astype(jnp.int32),
        ex, t2, p2, node2graph.astype(jnp.int32))
    return _tc_epilogue(part.reshape(NW * 28, 128))
